# baseline (speedup 1.0000x reference)
import functools

import jax
import jax.numpy as jnp
from jax.experimental import pallas as pl
from jax.experimental.pallas import tpu as pltpu

_SLOPE = 0.01


def _lrelu(v):
    return jnp.where(v >= 0, v, _SLOPE * v)


def _stage_body(x_ref, m_ref, b_ref, o_ref, xc_ref, *,
                Bel, D, H, W, Cin, Kb, K3, WoC, Dt):
    Ho = H // 2
    WC = W * Cin
    xc_ref[...] = jnp.zeros_like(xc_ref)
    v = x_ref[...].astype(xc_ref.dtype)
    for kh in range(3):
        hlo, hhi = max(0, 1 - kh), min(H, H + 1 - kh)
        xc_ref[:, pl.ds(1, D), pl.ds(hlo, hhi - hlo),
               pl.ds(kh * Kb + Cin, WC)] = v[:, :, hlo + kh - 1:hhi + kh - 1, :]
    for c in range(D // Dt):
        d0 = c * Dt
        acc = jnp.zeros((Bel * Dt * H, 2 * WoC), jnp.float32)
        for kd in range(3):
            a = xc_ref[:, pl.ds(d0 + kd, Dt), :, :].reshape(Bel * Dt * H, K3)
            acc = acc + jnp.dot(a, m_ref[kd],
                                preferred_element_type=jnp.float32)
        yw = jnp.maximum(acc[:, :WoC], acc[:, WoC:])
        y3 = yw.reshape(Bel * Dt, Ho, 2, WoC)
        yh = jnp.maximum(y3[:, :, 0, :], y3[:, :, 1, :])
        y4 = yh.reshape(Bel, Dt // 2, 2, Ho, WoC)
        yd = jnp.maximum(y4[:, :, 0], y4[:, :, 1])
        o_ref[:, pl.ds(d0 // 2, Dt // 2), :, :] = (
            _lrelu(yd + b_ref[...]).astype(o_ref.dtype))


def _stage_weights(w, W, dtype):
    Cout, Cin = w.shape[0], w.shape[1]
    Wp, Wo = W + 2, W // 2
    wt = jnp.transpose(w, (2, 3, 4, 1, 0))
    hot = (jnp.arange(Wp)[:, None, None, None]
           == (jnp.arange(3)[None, :, None, None]
               + jnp.arange(2)[None, None, :, None]
               + 2 * jnp.arange(Wo)[None, None, None, :])).astype(w.dtype)
    band = jnp.einsum('dhkio,pkbw->dhpibwo', wt, hot)
    return band.reshape(3, 3 * Wp * Cin, 2 * Wo * Cout).astype(dtype)


def _conv_stage(x, w, b, *, Bel, Dt):
    B, D, H, WC = x.shape
    Cout, Cin = w.shape[0], w.shape[1]
    W = WC // Cin
    Do, Ho, Wo = D // 2, H // 2, W // 2
    Kb = (W + 2) * Cin
    K3 = 3 * Kb
    WoC = Wo * Cout

    band = _stage_weights(w, W, jnp.bfloat16)
    blane = jnp.tile(b, Wo).reshape(1, 1, 1, WoC)

    body = functools.partial(_stage_body, Bel=Bel, D=D, H=H, W=W, Cin=Cin,
                             Kb=Kb, K3=K3, WoC=WoC, Dt=Dt)
    out = pl.pallas_call(
        body,
        out_shape=jax.ShapeDtypeStruct((B, Do, Ho, WoC), jnp.bfloat16),
        grid=(B // Bel,),
        in_specs=[
            pl.BlockSpec((Bel, D, H, WC), lambda i: (i, 0, 0, 0)),
            pl.BlockSpec((3, K3, 2 * WoC), lambda i: (0, 0, 0)),
            pl.BlockSpec((1, 1, 1, WoC), lambda i: (0, 0, 0, 0)),
        ],
        out_specs=pl.BlockSpec((Bel, Do, Ho, WoC), lambda i: (i, 0, 0, 0)),
        scratch_shapes=[
            pltpu.VMEM((Bel, D + 2, H, K3), jnp.bfloat16),
        ],
        compiler_params=pltpu.CompilerParams(
            dimension_semantics=("parallel",)),
    )(x, band, blane)
    return out


def _tail_body(x_ref, m4_ref, b4_ref, m5_ref, b5_ref,
               w1_ref, c1_ref, w2_ref, c2_ref, w3_ref, c3_ref, o_ref, *, Bt):
    a = x_ref[...].reshape(Bt * 4, 3 * 576)
    acc = jnp.dot(a, m4_ref[...], preferred_element_type=jnp.float32)
    t = jnp.maximum(acc[:, :256], acc[:, 256:])
    t = jnp.maximum(t[:, :128], t[:, 128:])
    t4 = t.reshape(Bt, 4, 128)
    td = jnp.maximum(t4[:, :2, :], t4[:, 2:, :])
    f = _lrelu(td + b4_ref[...])
    f5 = jnp.concatenate([f[:, 0, :], f[:, 1, :]], -1)
    y = jnp.dot(f5, m5_ref[...], preferred_element_type=jnp.float32)
    y = jnp.maximum(y[:, :256], y[:, 256:])
    y = jnp.maximum(y[:, :128], y[:, 128:])
    y = jnp.maximum(y[:, :64], y[:, 64:])
    feat = _lrelu(y + b5_ref[...])
    h = _lrelu(jnp.dot(feat, w1_ref[...],
                       preferred_element_type=jnp.float32) + c1_ref[...])
    h = _lrelu(jnp.dot(h, w2_ref[...],
                       preferred_element_type=jnp.float32) + c2_ref[...])
    out = jnp.dot(h, w3_ref[...],
                  preferred_element_type=jnp.float32) + c3_ref[...]
    o_ref[...] = out.reshape(Bt, 1, 24)


def _diag_cat(mats):
    rows = sum(m.shape[0] for m in mats)
    cols = sum(m.shape[1] for m in mats)
    out = jnp.zeros((rows, cols), mats[0].dtype)
    r = c = 0
    for m in mats:
        out = out.at[r:r + m.shape[0], c:c + m.shape[1]].set(m)
        r += m.shape[0]
        c += m.shape[1]
    return out


def _tail_weights(w4, b4, w5, b5, heads):
    wt4 = jnp.transpose(w4, (2, 3, 4, 1, 0))
    hot = (jnp.arange(6)[:, None, None, None]
           == (jnp.arange(3)[None, :, None, None]
               + jnp.arange(2)[None, None, :, None]
               + 2 * jnp.arange(2)[None, None, None, :])).astype(w4.dtype)
    M4 = jnp.einsum('dklio,pkbh,qlcw->dpqibchwo', wt4, hot, hot)
    M4 = M4.reshape(3 * 576, 512)
    b4l = jnp.tile(b4, 4).reshape(1, 1, 128)
    wt5 = jnp.transpose(w5, (2, 3, 4, 1, 0))
    off = jnp.arange(2)[:, None] - jnp.arange(2)[None, :] + 1
    M5 = wt5[off[:, None, None, :, None, None],
             off[None, :, None, None, :, None],
             off[None, None, :, None, None, :], :, :]
    M5 = jnp.transpose(M5, (0, 1, 2, 6, 3, 4, 5, 7)).reshape(256, 512)
    b5l = b5.reshape(1, 64)
    W1 = jnp.concatenate([p[0].T for p in heads], axis=1)
    c1 = jnp.concatenate([p[1] for p in heads]).reshape(1, 192)
    W2 = _diag_cat([p[2].T for p in heads])
    c2 = jnp.concatenate([p[3] for p in heads]).reshape(1, 96)
    W3 = _diag_cat([p[4].T for p in heads])
    c3 = jnp.concatenate([p[5] for p in heads]).reshape(1, 24)
    return M4, b4l, M5, b5l, W1, c1, W2, c2, W3, c3


def _tail(x3, w4, b4, w5, b5, heads):
    B = x3.shape[0]
    x3 = x3.astype(jnp.float32).reshape(B, 4, 4, 4, 16)
    xp = jnp.pad(x3, ((0, 0), (1, 1), (1, 1), (1, 1), (0, 0)))
    xp = xp.reshape(B, 6, 576)
    xcol = jnp.concatenate([xp[:, kd:kd + 4, :] for kd in range(3)], -1)
    xcol = xcol[:, jnp.array([0, 2, 1, 3]), :]
    mats = _tail_weights(w4, b4, w5, b5, heads)

    Bt = 128 if B % 128 == 0 else (B // 2 if B % 2 == 0 else B)
    out = pl.pallas_call(
        functools.partial(_tail_body, Bt=Bt),
        out_shape=jax.ShapeDtypeStruct((B, 1, 24), jnp.float32),
        grid=(B // Bt,),
        in_specs=[pl.BlockSpec((Bt, 4, 3 * 576), lambda i: (i, 0, 0))] + [
            pl.BlockSpec(m.shape, lambda i, n=m.ndim: (0,) * n)
            for m in mats],
        out_specs=pl.BlockSpec((Bt, 1, 24), lambda i: (i, 0, 0)),
        compiler_params=pltpu.CompilerParams(
            dimension_semantics=("parallel",)),
    )(xcol, *mats)
    out = out.reshape(B, 6, 4)
    return out[:, :3, :], out[:, 3:, :]


def kernel(x, conv0_w, conv0_b, conv1_w, conv1_b, conv2_w, conv2_b, conv3_w, conv3_b, conv4_w, conv4_b, head0_w1, head0_b1, head0_w2, head0_b2, head0_w3, head0_b3, head1_w1, head1_b1, head1_w2, head1_b2, head1_w3, head1_b3, head2_w1, head2_b1, head2_w2, head2_b2, head2_w3, head2_b3, head3_w1, head3_b1, head3_w2, head3_b2, head3_w3, head3_b3, head4_w1, head4_b1, head4_w2, head4_b2, head4_w3, head4_b3, head5_w1, head5_b1, head5_w2, head5_b2, head5_w3, head5_b3):
    B = x.shape[0]
    xc = x.reshape(B, 32, 32, 32)
    y0 = _conv_stage(xc, conv0_w, conv0_b, Bel=16, Dt=8)
    y1 = _conv_stage(y0, conv1_w, conv1_b, Bel=8, Dt=16)
    y2 = _conv_stage(y1, conv2_w, conv2_b, Bel=16, Dt=8)
    heads = [
        (head0_w1, head0_b1, head0_w2, head0_b2, head0_w3, head0_b3),
        (head1_w1, head1_b1, head1_w2, head1_b2, head1_w3, head1_b3),
        (head2_w1, head2_b1, head2_w2, head2_b2, head2_w3, head2_b3),
        (head3_w1, head3_b1, head3_w2, head3_b2, head3_w3, head3_b3),
        (head4_w1, head4_b1, head4_w2, head4_b2, head4_w3, head4_b3),
        (head5_w1, head5_b1, head5_w2, head5_b2, head5_w3, head5_b3),
    ]
    return _tail(y2, conv3_w, conv3_b, conv4_w, conv4_b, heads)


# speedup vs baseline: 1.0304x; 1.0304x over previous
import functools

import jax
import jax.numpy as jnp
from jax.experimental import pallas as pl
from jax.experimental.pallas import tpu as pltpu

_SLOPE = 0.01


def _lrelu(v):
    return jnp.where(v >= 0, v, _SLOPE * v)


def _stage_body(x_ref, m_ref, b_ref, o_ref, xc_ref, *,
                Bel, D, H, W, Cin, Kb, K3, WoC, Dt):
    Ho = H // 2
    WC = W * Cin
    xc_ref[...] = jnp.zeros_like(xc_ref)
    v = x_ref[...].astype(xc_ref.dtype)
    for kh in range(3):
        hlo, hhi = max(0, 1 - kh), min(H, H + 1 - kh)
        xc_ref[:, pl.ds(1, D), pl.ds(hlo, hhi - hlo),
               pl.ds(kh * Kb + Cin, WC)] = v[:, :, hlo + kh - 1:hhi + kh - 1, :]
    for c in range(D // Dt):
        d0 = c * Dt
        acc = jnp.zeros((Bel * Dt * H, 2 * WoC), jnp.float32)
        for kd in range(3):
            a = xc_ref[:, pl.ds(d0 + kd, Dt), :, :].reshape(Bel * Dt * H, K3)
            acc = acc + jnp.dot(a, m_ref[kd],
                                preferred_element_type=jnp.float32)
        yw = jnp.maximum(acc[:, :WoC], acc[:, WoC:])
        y3 = yw.reshape(Bel * Dt, Ho, 2, WoC)
        yh = jnp.maximum(y3[:, :, 0, :], y3[:, :, 1, :])
        y4 = yh.reshape(Bel, Dt // 2, 2, Ho, WoC)
        yd = jnp.maximum(y4[:, :, 0], y4[:, :, 1])
        o_ref[:, pl.ds(d0 // 2, Dt // 2), :, :] = (
            _lrelu(yd + b_ref[...]).astype(o_ref.dtype))


def _stage_weights(w, W, dtype):
    Cout, Cin = w.shape[0], w.shape[1]
    Wp, Wo = W + 2, W // 2
    wt = jnp.transpose(w, (2, 3, 4, 1, 0))
    hot = (jnp.arange(Wp)[:, None, None, None]
           == (jnp.arange(3)[None, :, None, None]
               + jnp.arange(2)[None, None, :, None]
               + 2 * jnp.arange(Wo)[None, None, None, :])).astype(w.dtype)
    band = jnp.einsum('dhkio,pkbw->dhpibwo', wt, hot)
    return band.reshape(3, 3 * Wp * Cin, 2 * Wo * Cout).astype(dtype)


def _conv_stage(x, w, b, *, Bel, Dt):
    B, D, H, WC = x.shape
    Cout, Cin = w.shape[0], w.shape[1]
    W = WC // Cin
    Do, Ho, Wo = D // 2, H // 2, W // 2
    Kb = (W + 2) * Cin
    K3 = 3 * Kb
    WoC = Wo * Cout

    band = _stage_weights(w, W, jnp.bfloat16)
    blane = jnp.tile(b, Wo).reshape(1, 1, 1, WoC)

    body = functools.partial(_stage_body, Bel=Bel, D=D, H=H, W=W, Cin=Cin,
                             Kb=Kb, K3=K3, WoC=WoC, Dt=Dt)
    out = pl.pallas_call(
        body,
        out_shape=jax.ShapeDtypeStruct((B, Do, Ho, WoC), jnp.bfloat16),
        grid=(B // Bel,),
        in_specs=[
            pl.BlockSpec((Bel, D, H, WC), lambda i: (i, 0, 0, 0)),
            pl.BlockSpec((3, K3, 2 * WoC), lambda i: (0, 0, 0)),
            pl.BlockSpec((1, 1, 1, WoC), lambda i: (0, 0, 0, 0)),
        ],
        out_specs=pl.BlockSpec((Bel, Do, Ho, WoC), lambda i: (i, 0, 0, 0)),
        scratch_shapes=[
            pltpu.VMEM((Bel, D + 2, H, K3), jnp.bfloat16),
        ],
        compiler_params=pltpu.CompilerParams(
            dimension_semantics=("parallel",)),
    )(x, band, blane)
    return out


def _tail_body(x_ref, m4_ref, b4_ref, m5_ref, b5_ref,
               w1_ref, c1_ref, w2_ref, c2_ref, w3_ref, c3_ref, o_ref, *, Bt):
    a = x_ref[...].reshape(Bt * 4, 3 * 576)
    acc = jnp.dot(a, m4_ref[...], preferred_element_type=jnp.float32)
    t = jnp.maximum(acc[:, :256], acc[:, 256:])
    t = jnp.maximum(t[:, :128], t[:, 128:])
    t4 = t.reshape(Bt, 4, 128)
    td = jnp.maximum(t4[:, :2, :], t4[:, 2:, :])
    f = _lrelu(td + b4_ref[...])
    f5 = jnp.concatenate([f[:, 0, :], f[:, 1, :]], -1)
    y = jnp.dot(f5, m5_ref[...], preferred_element_type=jnp.float32)
    y = jnp.maximum(y[:, :256], y[:, 256:])
    y = jnp.maximum(y[:, :128], y[:, 128:])
    y = jnp.maximum(y[:, :64], y[:, 64:])
    feat = _lrelu(y + b5_ref[...])
    h = _lrelu(jnp.dot(feat, w1_ref[...],
                       preferred_element_type=jnp.float32) + c1_ref[...])
    h = _lrelu(jnp.dot(h, w2_ref[...],
                       preferred_element_type=jnp.float32) + c2_ref[...])
    out = jnp.dot(h, w3_ref[...],
                  preferred_element_type=jnp.float32) + c3_ref[...]
    o_ref[...] = out.reshape(Bt, 1, 24)


def _diag_cat(mats):
    rows = sum(m.shape[0] for m in mats)
    cols = sum(m.shape[1] for m in mats)
    out = jnp.zeros((rows, cols), mats[0].dtype)
    r = c = 0
    for m in mats:
        out = out.at[r:r + m.shape[0], c:c + m.shape[1]].set(m)
        r += m.shape[0]
        c += m.shape[1]
    return out


def _tail_weights(w4, b4, w5, b5, heads):
    wt4 = jnp.transpose(w4, (2, 3, 4, 1, 0))
    hot = (jnp.arange(6)[:, None, None, None]
           == (jnp.arange(3)[None, :, None, None]
               + jnp.arange(2)[None, None, :, None]
               + 2 * jnp.arange(2)[None, None, None, :])).astype(w4.dtype)
    M4 = jnp.einsum('dklio,pkbh,qlcw->dpqibchwo', wt4, hot, hot)
    M4 = M4.reshape(3 * 576, 512)
    b4l = jnp.tile(b4, 4).reshape(1, 1, 128)
    wt5 = jnp.transpose(w5, (2, 3, 4, 1, 0))
    off = jnp.arange(2)[:, None] - jnp.arange(2)[None, :] + 1
    M5 = wt5[off[:, None, None, :, None, None],
             off[None, :, None, None, :, None],
             off[None, None, :, None, None, :], :, :]
    M5 = jnp.transpose(M5, (0, 1, 2, 6, 3, 4, 5, 7)).reshape(256, 512)
    b5l = b5.reshape(1, 64)
    W1 = jnp.concatenate([p[0].T for p in heads], axis=1)
    c1 = jnp.concatenate([p[1] for p in heads]).reshape(1, 192)
    W2 = _diag_cat([p[2].T for p in heads])
    c2 = jnp.concatenate([p[3] for p in heads]).reshape(1, 96)
    W3 = _diag_cat([p[4].T for p in heads])
    c3 = jnp.concatenate([p[5] for p in heads]).reshape(1, 24)
    return M4, b4l, M5, b5l, W1, c1, W2, c2, W3, c3


def _tail(x3, w4, b4, w5, b5, heads):
    B = x3.shape[0]
    x3 = x3.astype(jnp.float32).reshape(B, 4, 4, 4, 16)
    xp = jnp.pad(x3, ((0, 0), (1, 1), (1, 1), (1, 1), (0, 0)))
    xp = xp.reshape(B, 6, 576)
    xcol = jnp.concatenate([xp[:, kd:kd + 4, :] for kd in range(3)], -1)
    xcol = xcol[:, jnp.array([0, 2, 1, 3]), :]
    mats = _tail_weights(w4, b4, w5, b5, heads)

    Bt = 128 if B % 128 == 0 else (B // 2 if B % 2 == 0 else B)
    out = pl.pallas_call(
        functools.partial(_tail_body, Bt=Bt),
        out_shape=jax.ShapeDtypeStruct((B, 1, 24), jnp.float32),
        grid=(B // Bt,),
        in_specs=[pl.BlockSpec((Bt, 4, 3 * 576), lambda i: (i, 0, 0))] + [
            pl.BlockSpec(m.shape, lambda i, n=m.ndim: (0,) * n)
            for m in mats],
        out_specs=pl.BlockSpec((Bt, 1, 24), lambda i: (i, 0, 0)),
        compiler_params=pltpu.CompilerParams(
            dimension_semantics=("parallel",)),
    )(xcol, *mats)
    out = out.reshape(B, 6, 4)
    return out[:, :3, :], out[:, 3:, :]


def kernel(x, conv0_w, conv0_b, conv1_w, conv1_b, conv2_w, conv2_b, conv3_w, conv3_b, conv4_w, conv4_b, head0_w1, head0_b1, head0_w2, head0_b2, head0_w3, head0_b3, head1_w1, head1_b1, head1_w2, head1_b2, head1_w3, head1_b3, head2_w1, head2_b1, head2_w2, head2_b2, head2_w3, head2_b3, head3_w1, head3_b1, head3_w2, head3_b2, head3_w3, head3_b3, head4_w1, head4_b1, head4_w2, head4_b2, head4_w3, head4_b3, head5_w1, head5_b1, head5_w2, head5_b2, head5_w3, head5_b3):
    B = x.shape[0]
    xc = x.reshape(B, 32, 32, 32)
    y0 = _conv_stage(xc, conv0_w, conv0_b, Bel=8, Dt=8)
    y1 = _conv_stage(y0, conv1_w, conv1_b, Bel=8, Dt=16)
    y2 = _conv_stage(y1, conv2_w, conv2_b, Bel=16, Dt=8)
    heads = [
        (head0_w1, head0_b1, head0_w2, head0_b2, head0_w3, head0_b3),
        (head1_w1, head1_b1, head1_w2, head1_b2, head1_w3, head1_b3),
        (head2_w1, head2_b1, head2_w2, head2_b2, head2_w3, head2_b3),
        (head3_w1, head3_b1, head3_w2, head3_b2, head3_w3, head3_b3),
        (head4_w1, head4_b1, head4_w2, head4_b2, head4_w3, head4_b3),
        (head5_w1, head5_b1, head5_w2, head5_b2, head5_w3, head5_b3),
    ]
    return _tail(y2, conv3_w, conv3_b, conv4_w, conv4_b, heads)


# speedup vs baseline: 1.0505x; 1.0195x over previous
import functools

import jax
import jax.numpy as jnp
from jax.experimental import pallas as pl
from jax.experimental.pallas import tpu as pltpu

_SLOPE = 0.01


def _lrelu(v):
    return jnp.where(v >= 0, v, _SLOPE * v)


def _stage_body(x_ref, m_ref, b_ref, o_ref, xc_ref, *,
                Bel, D, H, W, Cin, Kb, K3, WoC, Dt):
    Ho = H // 2
    WC = W * Cin
    xc_ref[...] = jnp.zeros_like(xc_ref)
    v = x_ref[...].astype(xc_ref.dtype)
    for kh in range(3):
        hlo, hhi = max(0, 1 - kh), min(H, H + 1 - kh)
        xc_ref[:, pl.ds(1, D), pl.ds(hlo, hhi - hlo),
               pl.ds(kh * Kb + Cin, WC)] = v[:, :, hlo + kh - 1:hhi + kh - 1, :]
    for c in range(D // Dt):
        d0 = c * Dt
        acc = jnp.zeros((Bel * Dt * H, 2 * WoC), jnp.float32)
        for kd in range(3):
            a = xc_ref[:, pl.ds(d0 + kd, Dt), :, :].reshape(Bel * Dt * H, K3)
            acc = acc + jnp.dot(a, m_ref[kd],
                                preferred_element_type=jnp.float32)
        yw = jnp.maximum(acc[:, :WoC], acc[:, WoC:])
        y3 = yw.reshape(Bel * Dt, Ho, 2, WoC)
        yh = jnp.maximum(y3[:, :, 0, :], y3[:, :, 1, :])
        y4 = yh.reshape(Bel, Dt // 2, 2, Ho, WoC)
        yd = jnp.maximum(y4[:, :, 0], y4[:, :, 1])
        o_ref[:, pl.ds(d0 // 2, Dt // 2), :, :] = (
            _lrelu(yd + b_ref[...]).astype(o_ref.dtype))


def _stage_weights(w, W, dtype):
    Cout, Cin = w.shape[0], w.shape[1]
    Wp, Wo = W + 2, W // 2
    wt = jnp.transpose(w, (2, 3, 4, 1, 0))
    hot = (jnp.arange(Wp)[:, None, None, None]
           == (jnp.arange(3)[None, :, None, None]
               + jnp.arange(2)[None, None, :, None]
               + 2 * jnp.arange(Wo)[None, None, None, :])).astype(w.dtype)
    band = jnp.einsum('dhkio,pkbw->dhpibwo', wt, hot)
    return band.reshape(3, 3 * Wp * Cin, 2 * Wo * Cout).astype(dtype)


def _conv_stage(x, w, b, *, Bel, Dt):
    B, D, H, WC = x.shape
    Cout, Cin = w.shape[0], w.shape[1]
    W = WC // Cin
    Do, Ho, Wo = D // 2, H // 2, W // 2
    Kb = (W + 2) * Cin
    K3 = 3 * Kb
    WoC = Wo * Cout

    band = _stage_weights(w, W, jnp.bfloat16)
    blane = jnp.tile(b, Wo).reshape(1, 1, 1, WoC)

    body = functools.partial(_stage_body, Bel=Bel, D=D, H=H, W=W, Cin=Cin,
                             Kb=Kb, K3=K3, WoC=WoC, Dt=Dt)
    out = pl.pallas_call(
        body,
        out_shape=jax.ShapeDtypeStruct((B, Do, Ho, WoC), jnp.bfloat16),
        grid=(B // Bel,),
        in_specs=[
            pl.BlockSpec((Bel, D, H, WC), lambda i: (i, 0, 0, 0)),
            pl.BlockSpec((3, K3, 2 * WoC), lambda i: (0, 0, 0)),
            pl.BlockSpec((1, 1, 1, WoC), lambda i: (0, 0, 0, 0)),
        ],
        out_specs=pl.BlockSpec((Bel, Do, Ho, WoC), lambda i: (i, 0, 0, 0)),
        scratch_shapes=[
            pltpu.VMEM((Bel, D + 2, H, K3), jnp.bfloat16),
        ],
        compiler_params=pltpu.CompilerParams(
            dimension_semantics=("parallel",)),
    )(x, band, blane)
    return out


def _tail_body(x_ref, m4_ref, b4_ref, m5_ref, b5_ref,
               w1_ref, c1_ref, w2_ref, c2_ref, w3_ref, c3_ref, o_ref, *, Bt):
    a = x_ref[...].reshape(Bt * 4, 3 * 576)
    acc = jnp.dot(a, m4_ref[...], preferred_element_type=jnp.float32)
    t = jnp.maximum(acc[:, :256], acc[:, 256:])
    t = jnp.maximum(t[:, :128], t[:, 128:])
    t4 = t.reshape(Bt, 4, 128)
    td = jnp.maximum(t4[:, :2, :], t4[:, 2:, :])
    f = _lrelu(td + b4_ref[...])
    f5 = jnp.concatenate([f[:, 0, :], f[:, 1, :]], -1)
    y = jnp.dot(f5, m5_ref[...], preferred_element_type=jnp.float32)
    y = jnp.maximum(y[:, :256], y[:, 256:])
    y = jnp.maximum(y[:, :128], y[:, 128:])
    y = jnp.maximum(y[:, :64], y[:, 64:])
    feat = _lrelu(y + b5_ref[...])
    h = _lrelu(jnp.dot(feat, w1_ref[...],
                       preferred_element_type=jnp.float32) + c1_ref[...])
    h = _lrelu(jnp.dot(h, w2_ref[...],
                       preferred_element_type=jnp.float32) + c2_ref[...])
    out = jnp.dot(h, w3_ref[...],
                  preferred_element_type=jnp.float32) + c3_ref[...]
    o_ref[...] = out.reshape(Bt, 1, 24)


def _diag_cat(mats):
    rows = sum(m.shape[0] for m in mats)
    cols = sum(m.shape[1] for m in mats)
    out = jnp.zeros((rows, cols), mats[0].dtype)
    r = c = 0
    for m in mats:
        out = out.at[r:r + m.shape[0], c:c + m.shape[1]].set(m)
        r += m.shape[0]
        c += m.shape[1]
    return out


def _tail_weights(w4, b4, w5, b5, heads):
    wt4 = jnp.transpose(w4, (2, 3, 4, 1, 0))
    hot = (jnp.arange(6)[:, None, None, None]
           == (jnp.arange(3)[None, :, None, None]
               + jnp.arange(2)[None, None, :, None]
               + 2 * jnp.arange(2)[None, None, None, :])).astype(w4.dtype)
    M4 = jnp.einsum('dklio,pkbh,qlcw->dpqibchwo', wt4, hot, hot)
    M4 = M4.reshape(3 * 576, 512).astype(jnp.bfloat16)
    b4l = jnp.tile(b4, 4).reshape(1, 1, 128)
    wt5 = jnp.transpose(w5, (2, 3, 4, 1, 0))
    off = jnp.arange(2)[:, None] - jnp.arange(2)[None, :] + 1
    M5 = wt5[off[:, None, None, :, None, None],
             off[None, :, None, None, :, None],
             off[None, None, :, None, None, :], :, :]
    M5 = jnp.transpose(M5, (0, 1, 2, 6, 3, 4, 5, 7)).reshape(256, 512)
    b5l = b5.reshape(1, 64)
    W1 = jnp.concatenate([p[0].T for p in heads], axis=1)
    c1 = jnp.concatenate([p[1] for p in heads]).reshape(1, 192)
    W2 = _diag_cat([p[2].T for p in heads])
    c2 = jnp.concatenate([p[3] for p in heads]).reshape(1, 96)
    W3 = _diag_cat([p[4].T for p in heads])
    c3 = jnp.concatenate([p[5] for p in heads]).reshape(1, 24)
    return M4, b4l, M5, b5l, W1, c1, W2, c2, W3, c3


def _tail(x3, w4, b4, w5, b5, heads):
    B = x3.shape[0]
    x3 = x3.reshape(B, 4, 4, 4, 16)
    xp = jnp.pad(x3, ((0, 0), (1, 1), (1, 1), (1, 1), (0, 0)))
    xp = xp.reshape(B, 6, 576)
    xcol = jnp.concatenate([xp[:, kd:kd + 4, :] for kd in range(3)], -1)
    xcol = xcol[:, jnp.array([0, 2, 1, 3]), :]
    mats = _tail_weights(w4, b4, w5, b5, heads)

    Bt = 128 if B % 128 == 0 else (B // 2 if B % 2 == 0 else B)
    out = pl.pallas_call(
        functools.partial(_tail_body, Bt=Bt),
        out_shape=jax.ShapeDtypeStruct((B, 1, 24), jnp.float32),
        grid=(B // Bt,),
        in_specs=[pl.BlockSpec((Bt, 4, 3 * 576), lambda i: (i, 0, 0))] + [
            pl.BlockSpec(m.shape, lambda i, n=m.ndim: (0,) * n)
            for m in mats],
        out_specs=pl.BlockSpec((Bt, 1, 24), lambda i: (i, 0, 0)),
        compiler_params=pltpu.CompilerParams(
            dimension_semantics=("parallel",)),
    )(xcol, *mats)
    out = out.reshape(B, 6, 4)
    return out[:, :3, :], out[:, 3:, :]


def kernel(x, conv0_w, conv0_b, conv1_w, conv1_b, conv2_w, conv2_b, conv3_w, conv3_b, conv4_w, conv4_b, head0_w1, head0_b1, head0_w2, head0_b2, head0_w3, head0_b3, head1_w1, head1_b1, head1_w2, head1_b2, head1_w3, head1_b3, head2_w1, head2_b1, head2_w2, head2_b2, head2_w3, head2_b3, head3_w1, head3_b1, head3_w2, head3_b2, head3_w3, head3_b3, head4_w1, head4_b1, head4_w2, head4_b2, head4_w3, head4_b3, head5_w1, head5_b1, head5_w2, head5_b2, head5_w3, head5_b3):
    B = x.shape[0]
    xc = x.reshape(B, 32, 32, 32)
    y0 = _conv_stage(xc, conv0_w, conv0_b, Bel=8, Dt=8)
    y1 = _conv_stage(y0, conv1_w, conv1_b, Bel=8, Dt=16)
    y2 = _conv_stage(y1, conv2_w, conv2_b, Bel=16, Dt=8)
    heads = [
        (head0_w1, head0_b1, head0_w2, head0_b2, head0_w3, head0_b3),
        (head1_w1, head1_b1, head1_w2, head1_b2, head1_w3, head1_b3),
        (head2_w1, head2_b1, head2_w2, head2_b2, head2_w3, head2_b3),
        (head3_w1, head3_b1, head3_w2, head3_b2, head3_w3, head3_b3),
        (head4_w1, head4_b1, head4_w2, head4_b2, head4_w3, head4_b3),
        (head5_w1, head5_b1, head5_w2, head5_b2, head5_w3, head5_b3),
    ]
    return _tail(y2, conv3_w, conv3_b, conv4_w, conv4_b, heads)
